# initial kernel scaffold (unmeasured)
import functools

import jax
import jax.numpy as jnp
from jax import lax
from jax.experimental import pallas as pl
from jax.experimental.pallas import tpu as pltpu

N_DEV = 8


def kernel(A, B):
    m, k = A.shape
    _, n = B.shape

    def body(a_ref, b_ref, out_ref, mine_ref, inbox_ref, send_sems, recv_sems):
        my_pos = lax.axis_index("i")

        mine_ref[...] = jnp.dot(
            a_ref[...].astype(jnp.bfloat16),
            b_ref[...].astype(jnp.bfloat16),
            preferred_element_type=jnp.float32,
        )

        barrier_sem = pltpu.get_barrier_semaphore()
        for d in range(1, N_DEV):
            peer = (my_pos + d) % N_DEV
            pl.semaphore_signal(
                barrier_sem, inc=1,
                device_id=(peer,), device_id_type=pl.DeviceIdType.MESH,
            )
        pl.semaphore_wait(barrier_sem, N_DEV - 1)

        rdmas = []
        for d in range(1, N_DEV):
            peer = (my_pos + d) % N_DEV
            rdma = pltpu.make_async_remote_copy(
                src_ref=mine_ref,
                dst_ref=inbox_ref.at[d - 1],
                send_sem=send_sems.at[d - 1],
                recv_sem=recv_sems.at[d - 1],
                device_id=(peer,),
                device_id_type=pl.DeviceIdType.MESH,
            )
            rdma.start()
            rdmas.append(rdma)

        for rdma in rdmas:
            rdma.wait_send()

        total = mine_ref[...]
        for j in range(N_DEV - 1):
            recv = pltpu.make_async_remote_copy(
                src_ref=mine_ref,
                dst_ref=inbox_ref.at[j],
                send_sem=send_sems.at[j],
                recv_sem=recv_sems.at[j],
                device_id=(my_pos,),
                device_id_type=pl.DeviceIdType.MESH,
            )
            recv.wait_recv()
            total = total + inbox_ref[j]

        out_ref[...] = total / (1.0 + jnp.exp(-total))

        @functools.partial(
            pl.run_scoped, second_barrier=pltpu.SemaphoreType.REGULAR
        )
        def _(second_barrier):
            for d in range(1, N_DEV):
                peer = (my_pos + d) % N_DEV
                pl.semaphore_signal(
                    second_barrier, inc=1,
                    device_id=(peer,), device_id_type=pl.DeviceIdType.MESH,
                )
            pl.semaphore_wait(second_barrier, N_DEV - 1)

    return pl.pallas_call(
        body,
        out_shape=jax.ShapeDtypeStruct((m, n), jnp.float32),
        in_specs=[
            pl.BlockSpec(memory_space=pltpu.VMEM),
            pl.BlockSpec(memory_space=pltpu.VMEM),
        ],
        out_specs=pl.BlockSpec(memory_space=pltpu.VMEM),
        scratch_shapes=[
            pltpu.VMEM((m, n), jnp.float32),
            pltpu.VMEM((N_DEV - 1, m, n), jnp.float32),
            pltpu.SemaphoreType.DMA((N_DEV - 1,)),
            pltpu.SemaphoreType.DMA((N_DEV - 1,)),
        ],
        compiler_params=pltpu.CompilerParams(collective_id=0),
    )(A, B)


# baseline (device time: 19980 ns/iter reference)
import functools

import jax
import jax.numpy as jnp
from jax import lax
from jax.experimental import pallas as pl
from jax.experimental.pallas import tpu as pltpu

N_DEV = 8


def kernel(A, B):
    m, k = A.shape
    _, n = B.shape
    chunk = m // N_DEV

    def body(
        a_ref, b_ref, out_ref,
        partial_ref, mine_ref, ag_send_ref, rs_inbox, ag_inbox,
        rs_send_sems, rs_recv_sems, ag_send_sems, ag_recv_sems,
    ):
        my_pos = lax.axis_index("i")

        partial = jnp.dot(
            a_ref[...].astype(jnp.bfloat16),
            b_ref[...].astype(jnp.bfloat16),
            preferred_element_type=jnp.float32,
        )
        partial_ref[...] = partial
        mine_ref[...] = partial.astype(jnp.bfloat16)

        barrier_sem = pltpu.get_barrier_semaphore()
        for d in range(1, N_DEV):
            peer = (my_pos + d) % N_DEV
            pl.semaphore_signal(
                barrier_sem, inc=1,
                device_id=(peer,), device_id_type=pl.DeviceIdType.MESH,
            )
        pl.semaphore_wait(barrier_sem, N_DEV - 1)

        rs_rdmas = []
        for d in range(1, N_DEV):
            peer = (my_pos + d) % N_DEV
            rdma = pltpu.make_async_remote_copy(
                src_ref=mine_ref.at[pl.ds(peer * chunk, chunk)],
                dst_ref=rs_inbox.at[d - 1],
                send_sem=rs_send_sems.at[d - 1],
                recv_sem=rs_recv_sems.at[d - 1],
                device_id=(peer,),
                device_id_type=pl.DeviceIdType.MESH,
            )
            rdma.start()
            rs_rdmas.append(rdma)

        acc = partial_ref[pl.ds(my_pos * chunk, chunk), :]
        for j in range(N_DEV - 1):
            recv = pltpu.make_async_remote_copy(
                src_ref=rs_inbox.at[j],
                dst_ref=rs_inbox.at[j],
                send_sem=rs_send_sems.at[j],
                recv_sem=rs_recv_sems.at[j],
                device_id=(my_pos,),
                device_id_type=pl.DeviceIdType.MESH,
            )
            recv.wait_recv()
            acc = acc + rs_inbox[j].astype(jnp.float32)

        z = acc
        my_block = z / (1.0 + jnp.exp(-z))
        out_ref[pl.ds(my_pos * chunk, chunk), :] = my_block
        ag_send_ref[...] = my_block.astype(jnp.bfloat16)

        ag_rdmas = []
        for d in range(1, N_DEV):
            peer = (my_pos + d) % N_DEV
            rdma = pltpu.make_async_remote_copy(
                src_ref=ag_send_ref,
                dst_ref=ag_inbox.at[d - 1],
                send_sem=ag_send_sems.at[d - 1],
                recv_sem=ag_recv_sems.at[d - 1],
                device_id=(peer,),
                device_id_type=pl.DeviceIdType.MESH,
            )
            rdma.start()
            ag_rdmas.append(rdma)

        for j in range(N_DEV - 1):
            recv = pltpu.make_async_remote_copy(
                src_ref=ag_inbox.at[j],
                dst_ref=ag_inbox.at[j],
                send_sem=ag_send_sems.at[j],
                recv_sem=ag_recv_sems.at[j],
                device_id=(my_pos,),
                device_id_type=pl.DeviceIdType.MESH,
            )
            recv.wait_recv()
            src_pos = (my_pos - j - 1) % N_DEV
            out_ref[pl.ds(src_pos * chunk, chunk), :] = (
                ag_inbox[j].astype(jnp.float32)
            )

        for rdma in rs_rdmas:
            rdma.wait_send()
        for rdma in ag_rdmas:
            rdma.wait_send()

        @functools.partial(
            pl.run_scoped, second_barrier=pltpu.SemaphoreType.REGULAR
        )
        def _(second_barrier):
            for d in range(1, N_DEV):
                peer = (my_pos + d) % N_DEV
                pl.semaphore_signal(
                    second_barrier, inc=1,
                    device_id=(peer,), device_id_type=pl.DeviceIdType.MESH,
                )
            pl.semaphore_wait(second_barrier, N_DEV - 1)

    return pl.pallas_call(
        body,
        out_shape=jax.ShapeDtypeStruct((m, n), jnp.float32),
        in_specs=[
            pl.BlockSpec(memory_space=pltpu.VMEM),
            pl.BlockSpec(memory_space=pltpu.VMEM),
        ],
        out_specs=pl.BlockSpec(memory_space=pltpu.VMEM),
        scratch_shapes=[
            pltpu.VMEM((m, n), jnp.float32),
            pltpu.VMEM((m, n), jnp.bfloat16),
            pltpu.VMEM((chunk, n), jnp.bfloat16),
            pltpu.VMEM((N_DEV - 1, chunk, n), jnp.bfloat16),
            pltpu.VMEM((N_DEV - 1, chunk, n), jnp.bfloat16),
            pltpu.SemaphoreType.DMA((N_DEV - 1,)),
            pltpu.SemaphoreType.DMA((N_DEV - 1,)),
            pltpu.SemaphoreType.DMA((N_DEV - 1,)),
            pltpu.SemaphoreType.DMA((N_DEV - 1,)),
        ],
        compiler_params=pltpu.CompilerParams(collective_id=0),
    )(A, B)


# device time: 19584 ns/iter; 1.0202x vs baseline; 1.0202x over previous
import functools

import jax
import jax.numpy as jnp
from jax import lax
from jax.experimental import pallas as pl
from jax.experimental.pallas import tpu as pltpu

N_DEV = 8


def kernel(A, B):
    m, k = A.shape
    _, n = B.shape
    chunk = m // N_DEV

    def body(
        a_ref, b_ref, out_ref,
        partial_ref, mine_ref, ag_send_ref, rs_inbox, ag_inbox,
        rs_send_sems, rs_recv_sems, ag_send_sems, ag_recv_sems,
    ):
        my_pos = lax.axis_index("i")

        barrier_sem = pltpu.get_barrier_semaphore()
        for d in range(1, N_DEV):
            peer = (my_pos + d) % N_DEV
            pl.semaphore_signal(
                barrier_sem, inc=1,
                device_id=(peer,), device_id_type=pl.DeviceIdType.MESH,
            )

        partial = jnp.dot(
            a_ref[...].astype(jnp.bfloat16),
            b_ref[...].astype(jnp.bfloat16),
            preferred_element_type=jnp.float32,
        )
        partial_ref[...] = partial
        mine_ref[...] = partial.astype(jnp.bfloat16)

        pl.semaphore_wait(barrier_sem, N_DEV - 1)

        rs_rdmas = []
        for d in range(1, N_DEV):
            peer = (my_pos + d) % N_DEV
            rdma = pltpu.make_async_remote_copy(
                src_ref=mine_ref.at[pl.ds(peer * chunk, chunk)],
                dst_ref=rs_inbox.at[d - 1],
                send_sem=rs_send_sems.at[d - 1],
                recv_sem=rs_recv_sems.at[d - 1],
                device_id=(peer,),
                device_id_type=pl.DeviceIdType.MESH,
            )
            rdma.start()
            rs_rdmas.append(rdma)

        acc = partial_ref[pl.ds(my_pos * chunk, chunk), :]
        for j in range(N_DEV - 1):
            recv = pltpu.make_async_remote_copy(
                src_ref=rs_inbox.at[j],
                dst_ref=rs_inbox.at[j],
                send_sem=rs_send_sems.at[j],
                recv_sem=rs_recv_sems.at[j],
                device_id=(my_pos,),
                device_id_type=pl.DeviceIdType.MESH,
            )
            recv.wait_recv()
            acc = acc + rs_inbox[j].astype(jnp.float32)

        z = acc
        my_block = z / (1.0 + jnp.exp(-z))
        out_ref[pl.ds(my_pos * chunk, chunk), :] = my_block
        ag_send_ref[...] = my_block.astype(jnp.bfloat16)

        ag_rdmas = []
        for d in range(1, N_DEV):
            peer = (my_pos + d) % N_DEV
            rdma = pltpu.make_async_remote_copy(
                src_ref=ag_send_ref,
                dst_ref=ag_inbox.at[d - 1],
                send_sem=ag_send_sems.at[d - 1],
                recv_sem=ag_recv_sems.at[d - 1],
                device_id=(peer,),
                device_id_type=pl.DeviceIdType.MESH,
            )
            rdma.start()
            ag_rdmas.append(rdma)

        for j in range(N_DEV - 1):
            recv = pltpu.make_async_remote_copy(
                src_ref=ag_inbox.at[j],
                dst_ref=ag_inbox.at[j],
                send_sem=ag_send_sems.at[j],
                recv_sem=ag_recv_sems.at[j],
                device_id=(my_pos,),
                device_id_type=pl.DeviceIdType.MESH,
            )
            recv.wait_recv()
            src_pos = (my_pos - j - 1) % N_DEV
            out_ref[pl.ds(src_pos * chunk, chunk), :] = (
                ag_inbox[j].astype(jnp.float32)
            )

        for rdma in rs_rdmas:
            rdma.wait_send()
        for rdma in ag_rdmas:
            rdma.wait_send()

        @functools.partial(
            pl.run_scoped, second_barrier=pltpu.SemaphoreType.REGULAR
        )
        def _(second_barrier):
            for d in range(1, N_DEV):
                peer = (my_pos + d) % N_DEV
                pl.semaphore_signal(
                    second_barrier, inc=1,
                    device_id=(peer,), device_id_type=pl.DeviceIdType.MESH,
                )
            pl.semaphore_wait(second_barrier, N_DEV - 1)

    return pl.pallas_call(
        body,
        out_shape=jax.ShapeDtypeStruct((m, n), jnp.float32),
        in_specs=[
            pl.BlockSpec(memory_space=pltpu.VMEM),
            pl.BlockSpec(memory_space=pltpu.VMEM),
        ],
        out_specs=pl.BlockSpec(memory_space=pltpu.VMEM),
        scratch_shapes=[
            pltpu.VMEM((m, n), jnp.float32),
            pltpu.VMEM((m, n), jnp.bfloat16),
            pltpu.VMEM((chunk, n), jnp.bfloat16),
            pltpu.VMEM((N_DEV - 1, chunk, n), jnp.bfloat16),
            pltpu.VMEM((N_DEV - 1, chunk, n), jnp.bfloat16),
            pltpu.SemaphoreType.DMA((N_DEV - 1,)),
            pltpu.SemaphoreType.DMA((N_DEV - 1,)),
            pltpu.SemaphoreType.DMA((N_DEV - 1,)),
            pltpu.SemaphoreType.DMA((N_DEV - 1,)),
        ],
        compiler_params=pltpu.CompilerParams(collective_id=0),
    )(A, B)


# device time: 18592 ns/iter; 1.0747x vs baseline; 1.0534x over previous
import functools

import jax
import jax.numpy as jnp
from jax import lax
from jax.experimental import pallas as pl
from jax.experimental.pallas import tpu as pltpu

N_DEV = 8
N_HALF = 2


def kernel(A, B):
    m, k = A.shape
    _, n = B.shape
    chunk = m // N_DEV
    half = n // N_HALF

    def body(
        a_ref, b_ref, out_ref,
        partial_ref, mine_ref, ag_send_ref, rs_inbox, ag_inbox,
        rs_send_sems, rs_recv_sems, ag_send_sems, ag_recv_sems,
    ):
        my_pos = lax.axis_index("i")

        barrier_sem = pltpu.get_barrier_semaphore()
        for d in range(1, N_DEV):
            peer = (my_pos + d) % N_DEV
            pl.semaphore_signal(
                barrier_sem, inc=1,
                device_id=(peer,), device_id_type=pl.DeviceIdType.MESH,
            )

        partial = jnp.dot(
            a_ref[...].astype(jnp.bfloat16),
            b_ref[...].astype(jnp.bfloat16),
            preferred_element_type=jnp.float32,
        )
        partial_ref[...] = partial
        for h in range(N_HALF):
            mine_ref[h] = partial[:, h * half:(h + 1) * half].astype(
                jnp.bfloat16
            )

        pl.semaphore_wait(barrier_sem, N_DEV - 1)

        rs_rdmas = []
        for h in range(N_HALF):
            for d in range(1, N_DEV):
                peer = (my_pos + d) % N_DEV
                rdma = pltpu.make_async_remote_copy(
                    src_ref=mine_ref.at[h, pl.ds(peer * chunk, chunk)],
                    dst_ref=rs_inbox.at[h, d - 1],
                    send_sem=rs_send_sems.at[h, d - 1],
                    recv_sem=rs_recv_sems.at[h, d - 1],
                    device_id=(peer,),
                    device_id_type=pl.DeviceIdType.MESH,
                )
                rdma.start()
                rs_rdmas.append(rdma)

        ag_rdmas = []
        for h in range(N_HALF):
            acc = partial_ref[
                pl.ds(my_pos * chunk, chunk), h * half:(h + 1) * half
            ]
            for j in range(N_DEV - 1):
                recv = pltpu.make_async_remote_copy(
                    src_ref=rs_inbox.at[h, j],
                    dst_ref=rs_inbox.at[h, j],
                    send_sem=rs_send_sems.at[h, j],
                    recv_sem=rs_recv_sems.at[h, j],
                    device_id=(my_pos,),
                    device_id_type=pl.DeviceIdType.MESH,
                )
                recv.wait_recv()
                acc = acc + rs_inbox[h, j].astype(jnp.float32)

            my_block = acc / (1.0 + jnp.exp(-acc))
            out_ref[
                pl.ds(my_pos * chunk, chunk), h * half:(h + 1) * half
            ] = my_block
            ag_send_ref[h] = my_block.astype(jnp.bfloat16)

            for d in range(1, N_DEV):
                peer = (my_pos + d) % N_DEV
                rdma = pltpu.make_async_remote_copy(
                    src_ref=ag_send_ref.at[h],
                    dst_ref=ag_inbox.at[h, d - 1],
                    send_sem=ag_send_sems.at[h, d - 1],
                    recv_sem=ag_recv_sems.at[h, d - 1],
                    device_id=(peer,),
                    device_id_type=pl.DeviceIdType.MESH,
                )
                rdma.start()
                ag_rdmas.append(rdma)

        for h in range(N_HALF):
            for j in range(N_DEV - 1):
                recv = pltpu.make_async_remote_copy(
                    src_ref=ag_inbox.at[h, j],
                    dst_ref=ag_inbox.at[h, j],
                    send_sem=ag_send_sems.at[h, j],
                    recv_sem=ag_recv_sems.at[h, j],
                    device_id=(my_pos,),
                    device_id_type=pl.DeviceIdType.MESH,
                )
                recv.wait_recv()
                src_pos = (my_pos - j - 1) % N_DEV
                out_ref[
                    pl.ds(src_pos * chunk, chunk), h * half:(h + 1) * half
                ] = ag_inbox[h, j].astype(jnp.float32)

        for rdma in rs_rdmas:
            rdma.wait_send()
        for rdma in ag_rdmas:
            rdma.wait_send()

        @functools.partial(
            pl.run_scoped, second_barrier=pltpu.SemaphoreType.REGULAR
        )
        def _(second_barrier):
            for d in range(1, N_DEV):
                peer = (my_pos + d) % N_DEV
                pl.semaphore_signal(
                    second_barrier, inc=1,
                    device_id=(peer,), device_id_type=pl.DeviceIdType.MESH,
                )
            pl.semaphore_wait(second_barrier, N_DEV - 1)

    return pl.pallas_call(
        body,
        out_shape=jax.ShapeDtypeStruct((m, n), jnp.float32),
        in_specs=[
            pl.BlockSpec(memory_space=pltpu.VMEM),
            pl.BlockSpec(memory_space=pltpu.VMEM),
        ],
        out_specs=pl.BlockSpec(memory_space=pltpu.VMEM),
        scratch_shapes=[
            pltpu.VMEM((m, n), jnp.float32),
            pltpu.VMEM((N_HALF, m, half), jnp.bfloat16),
            pltpu.VMEM((N_HALF, chunk, half), jnp.bfloat16),
            pltpu.VMEM((N_HALF, N_DEV - 1, chunk, half), jnp.bfloat16),
            pltpu.VMEM((N_HALF, N_DEV - 1, chunk, half), jnp.bfloat16),
            pltpu.SemaphoreType.DMA((N_HALF, N_DEV - 1)),
            pltpu.SemaphoreType.DMA((N_HALF, N_DEV - 1)),
            pltpu.SemaphoreType.DMA((N_HALF, N_DEV - 1)),
            pltpu.SemaphoreType.DMA((N_HALF, N_DEV - 1)),
        ],
        compiler_params=pltpu.CompilerParams(collective_id=0),
    )(A, B)


# device time: 17559 ns/iter; 1.1379x vs baseline; 1.0588x over previous
import functools

import jax
import jax.numpy as jnp
from jax import lax
from jax.experimental import pallas as pl
from jax.experimental.pallas import tpu as pltpu

N_DEV = 8
N_HALF = 2


def kernel(A, B):
    m, k = A.shape
    _, n = B.shape
    chunk = m // N_DEV
    half = n // N_HALF

    def body(
        a_ref, b_ref, out_ref,
        mine_ref, ag_send_ref, rs_inbox, ag_inbox,
        rs_send_sems, rs_recv_sems, ag_send_sems, ag_recv_sems,
    ):
        my_pos = lax.axis_index("i")

        barrier_sem = pltpu.get_barrier_semaphore()
        for d in range(1, N_DEV):
            peer = (my_pos + d) % N_DEV
            pl.semaphore_signal(
                barrier_sem, inc=1,
                device_id=(peer,), device_id_type=pl.DeviceIdType.MESH,
            )

        partial = jnp.dot(
            a_ref[...].astype(jnp.bfloat16),
            b_ref[...].astype(jnp.bfloat16),
            preferred_element_type=jnp.float32,
        )
        for h in range(N_HALF):
            mine_ref[h] = partial[:, h * half:(h + 1) * half].astype(
                jnp.bfloat16
            )

        pl.semaphore_wait(barrier_sem, N_DEV - 1)

        rs_rdmas = []
        for h in range(N_HALF):
            for d in range(1, N_DEV):
                peer = (my_pos + d) % N_DEV
                rdma = pltpu.make_async_remote_copy(
                    src_ref=mine_ref.at[h, pl.ds(peer * chunk, chunk)],
                    dst_ref=rs_inbox.at[h, d - 1],
                    send_sem=rs_send_sems.at[h, d - 1],
                    recv_sem=rs_recv_sems.at[h, d - 1],
                    device_id=(peer,),
                    device_id_type=pl.DeviceIdType.MESH,
                )
                rdma.start()
                rs_rdmas.append(rdma)

        ag_rdmas = []
        for h in range(N_HALF):
            acc = mine_ref[h, pl.ds(my_pos * chunk, chunk), :].astype(
                jnp.float32
            )
            for j in range(N_DEV - 1):
                recv = pltpu.make_async_remote_copy(
                    src_ref=rs_inbox.at[h, j],
                    dst_ref=rs_inbox.at[h, j],
                    send_sem=rs_send_sems.at[h, j],
                    recv_sem=rs_recv_sems.at[h, j],
                    device_id=(my_pos,),
                    device_id_type=pl.DeviceIdType.MESH,
                )
                recv.wait_recv()
                acc = acc + rs_inbox[h, j].astype(jnp.float32)

            my_block = acc / (1.0 + jnp.exp(-acc))
            out_ref[
                pl.ds(my_pos * chunk, chunk), h * half:(h + 1) * half
            ] = my_block
            ag_send_ref[h] = my_block.astype(jnp.bfloat16)

            for d in range(1, N_DEV):
                peer = (my_pos + d) % N_DEV
                rdma = pltpu.make_async_remote_copy(
                    src_ref=ag_send_ref.at[h],
                    dst_ref=ag_inbox.at[h, d - 1],
                    send_sem=ag_send_sems.at[h, d - 1],
                    recv_sem=ag_recv_sems.at[h, d - 1],
                    device_id=(peer,),
                    device_id_type=pl.DeviceIdType.MESH,
                )
                rdma.start()
                ag_rdmas.append(rdma)

        for h in range(N_HALF):
            for j in range(N_DEV - 1):
                recv = pltpu.make_async_remote_copy(
                    src_ref=ag_inbox.at[h, j],
                    dst_ref=ag_inbox.at[h, j],
                    send_sem=ag_send_sems.at[h, j],
                    recv_sem=ag_recv_sems.at[h, j],
                    device_id=(my_pos,),
                    device_id_type=pl.DeviceIdType.MESH,
                )
                recv.wait_recv()
                src_pos = (my_pos - j - 1) % N_DEV
                out_ref[
                    pl.ds(src_pos * chunk, chunk), h * half:(h + 1) * half
                ] = ag_inbox[h, j].astype(jnp.float32)

        for rdma in rs_rdmas:
            rdma.wait_send()
        for rdma in ag_rdmas:
            rdma.wait_send()


    return pl.pallas_call(
        body,
        out_shape=jax.ShapeDtypeStruct((m, n), jnp.float32),
        in_specs=[
            pl.BlockSpec(memory_space=pltpu.VMEM),
            pl.BlockSpec(memory_space=pltpu.VMEM),
        ],
        out_specs=pl.BlockSpec(memory_space=pltpu.VMEM),
        scratch_shapes=[
            pltpu.VMEM((N_HALF, m, half), jnp.bfloat16),
            pltpu.VMEM((N_HALF, chunk, half), jnp.bfloat16),
            pltpu.VMEM((N_HALF, N_DEV - 1, chunk, half), jnp.bfloat16),
            pltpu.VMEM((N_HALF, N_DEV - 1, chunk, half), jnp.bfloat16),
            pltpu.SemaphoreType.DMA((N_HALF, N_DEV - 1)),
            pltpu.SemaphoreType.DMA((N_HALF, N_DEV - 1)),
            pltpu.SemaphoreType.DMA((N_HALF, N_DEV - 1)),
            pltpu.SemaphoreType.DMA((N_HALF, N_DEV - 1)),
        ],
        compiler_params=pltpu.CompilerParams(collective_id=0),
    )(A, B)


# device time: 17362 ns/iter; 1.1508x vs baseline; 1.0113x over previous
import functools

import jax
import jax.numpy as jnp
from jax import lax
from jax.experimental import pallas as pl
from jax.experimental.pallas import tpu as pltpu

N_DEV = 8
N_HALF = 2


def kernel(A, B):
    m, k = A.shape
    _, n = B.shape
    chunk = m // N_DEV
    half = n // N_HALF

    def body(
        a_ref, b_ref, out_ref,
        mine_ref, rs_inbox,
        rs_send_sems, rs_recv_sems, ag_send_sems, ag_recv_sems,
    ):
        my_pos = lax.axis_index("i")

        barrier_sem = pltpu.get_barrier_semaphore()
        for d in range(1, N_DEV):
            peer = (my_pos + d) % N_DEV
            pl.semaphore_signal(
                barrier_sem, inc=1,
                device_id=(peer,), device_id_type=pl.DeviceIdType.MESH,
            )

        partial = jnp.dot(
            a_ref[...].astype(jnp.bfloat16),
            b_ref[...].astype(jnp.bfloat16),
            preferred_element_type=jnp.float32,
        )
        for h in range(N_HALF):
            mine_ref[h] = partial[:, h * half:(h + 1) * half].astype(
                jnp.bfloat16
            )

        pl.semaphore_wait(barrier_sem, N_DEV - 1)

        rs_rdmas = []
        for h in range(N_HALF):
            for d in range(1, N_DEV):
                peer = (my_pos + d) % N_DEV
                rdma = pltpu.make_async_remote_copy(
                    src_ref=mine_ref.at[h, pl.ds(peer * chunk, chunk)],
                    dst_ref=rs_inbox.at[h, d - 1],
                    send_sem=rs_send_sems.at[h, d - 1],
                    recv_sem=rs_recv_sems.at[h, d - 1],
                    device_id=(peer,),
                    device_id_type=pl.DeviceIdType.MESH,
                )
                rdma.start()
                rs_rdmas.append(rdma)

        ag_rdmas = []
        for h in range(N_HALF):
            acc = mine_ref[h, pl.ds(my_pos * chunk, chunk), :].astype(
                jnp.float32
            )
            for j in range(N_DEV - 1):
                recv = pltpu.make_async_remote_copy(
                    src_ref=rs_inbox.at[h, j],
                    dst_ref=rs_inbox.at[h, j],
                    send_sem=rs_send_sems.at[h, j],
                    recv_sem=rs_recv_sems.at[h, j],
                    device_id=(my_pos,),
                    device_id_type=pl.DeviceIdType.MESH,
                )
                recv.wait_recv()
                acc = acc + rs_inbox[h, j].astype(jnp.float32)

            my_block = acc / (1.0 + jnp.exp(-acc))
            out_ref[
                pl.ds(my_pos * chunk, chunk), h * half:(h + 1) * half
            ] = my_block.astype(jnp.bfloat16)

            for d in range(1, N_DEV):
                peer = (my_pos + d) % N_DEV
                rdma = pltpu.make_async_remote_copy(
                    src_ref=out_ref.at[
                        pl.ds(my_pos * chunk, chunk),
                        pl.ds(h * half, half),
                    ],
                    dst_ref=out_ref.at[
                        pl.ds(my_pos * chunk, chunk),
                        pl.ds(h * half, half),
                    ],
                    send_sem=ag_send_sems.at[h, d - 1],
                    recv_sem=ag_recv_sems.at[h, d - 1],
                    device_id=(peer,),
                    device_id_type=pl.DeviceIdType.MESH,
                )
                rdma.start()
                ag_rdmas.append(rdma)

        for h in range(N_HALF):
            for j in range(N_DEV - 1):
                src_pos = (my_pos - j - 1) % N_DEV
                recv = pltpu.make_async_remote_copy(
                    src_ref=out_ref.at[
                        pl.ds(src_pos * chunk, chunk),
                        pl.ds(h * half, half),
                    ],
                    dst_ref=out_ref.at[
                        pl.ds(src_pos * chunk, chunk),
                        pl.ds(h * half, half),
                    ],
                    send_sem=ag_send_sems.at[h, j],
                    recv_sem=ag_recv_sems.at[h, j],
                    device_id=(my_pos,),
                    device_id_type=pl.DeviceIdType.MESH,
                )
                recv.wait_recv()

        for rdma in rs_rdmas:
            rdma.wait_send()
        for rdma in ag_rdmas:
            rdma.wait_send()


    return pl.pallas_call(
        body,
        out_shape=jax.ShapeDtypeStruct((m, n), jnp.bfloat16),
        in_specs=[
            pl.BlockSpec(memory_space=pltpu.VMEM),
            pl.BlockSpec(memory_space=pltpu.VMEM),
        ],
        out_specs=pl.BlockSpec(memory_space=pltpu.VMEM),
        scratch_shapes=[
            pltpu.VMEM((N_HALF, m, half), jnp.bfloat16),
            pltpu.VMEM((N_HALF, N_DEV - 1, chunk, half), jnp.bfloat16),
            pltpu.SemaphoreType.DMA((N_HALF, N_DEV - 1)),
            pltpu.SemaphoreType.DMA((N_HALF, N_DEV - 1)),
            pltpu.SemaphoreType.DMA((N_HALF, N_DEV - 1)),
            pltpu.SemaphoreType.DMA((N_HALF, N_DEV - 1)),
        ],
        compiler_params=pltpu.CompilerParams(collective_id=0),
    )(A, B)


# device time: 16979 ns/iter; 1.1767x vs baseline; 1.0226x over previous
import functools

import jax
import jax.numpy as jnp
from jax import lax
from jax.experimental import pallas as pl
from jax.experimental.pallas import tpu as pltpu

N_DEV = 8
N_HALF = 2
_SEND_ORDER = (4, 3, 5, 2, 6, 1, 7)


def kernel(A, B):
    m, k = A.shape
    _, n = B.shape
    chunk = m // N_DEV
    half = n // N_HALF

    def body(
        a_ref, b_ref, out_ref,
        mine_ref, rs_inbox,
        rs_send_sems, rs_recv_sems, ag_send_sems, ag_recv_sems,
    ):
        my_pos = lax.axis_index("i")

        barrier_sem = pltpu.get_barrier_semaphore()
        for d in range(1, N_DEV):
            peer = (my_pos + d) % N_DEV
            pl.semaphore_signal(
                barrier_sem, inc=1,
                device_id=(peer,), device_id_type=pl.DeviceIdType.MESH,
            )

        partial = jnp.dot(
            a_ref[...].astype(jnp.bfloat16),
            b_ref[...].astype(jnp.bfloat16),
            preferred_element_type=jnp.float32,
        )
        for h in range(N_HALF):
            mine_ref[h] = partial[:, h * half:(h + 1) * half].astype(
                jnp.bfloat16
            )

        pl.semaphore_wait(barrier_sem, N_DEV - 1)

        rs_rdmas = []
        for h in range(N_HALF):
            for d in _SEND_ORDER:
                peer = (my_pos + d) % N_DEV
                rdma = pltpu.make_async_remote_copy(
                    src_ref=mine_ref.at[h, pl.ds(peer * chunk, chunk)],
                    dst_ref=rs_inbox.at[h, d - 1],
                    send_sem=rs_send_sems.at[h, d - 1],
                    recv_sem=rs_recv_sems.at[h, d - 1],
                    device_id=(peer,),
                    device_id_type=pl.DeviceIdType.MESH,
                )
                rdma.start()
                rs_rdmas.append(rdma)

        ag_rdmas = []
        for h in range(N_HALF):
            acc = mine_ref[h, pl.ds(my_pos * chunk, chunk), :].astype(
                jnp.float32
            )
            for j in range(N_DEV - 1):
                recv = pltpu.make_async_remote_copy(
                    src_ref=rs_inbox.at[h, j],
                    dst_ref=rs_inbox.at[h, j],
                    send_sem=rs_send_sems.at[h, j],
                    recv_sem=rs_recv_sems.at[h, j],
                    device_id=(my_pos,),
                    device_id_type=pl.DeviceIdType.MESH,
                )
                recv.wait_recv()
                acc = acc + rs_inbox[h, j].astype(jnp.float32)

            my_block = acc / (1.0 + jnp.exp(-acc))
            out_ref[
                pl.ds(my_pos * chunk, chunk), h * half:(h + 1) * half
            ] = my_block.astype(jnp.bfloat16)

            for d in _SEND_ORDER:
                peer = (my_pos + d) % N_DEV
                rdma = pltpu.make_async_remote_copy(
                    src_ref=out_ref.at[
                        pl.ds(my_pos * chunk, chunk),
                        pl.ds(h * half, half),
                    ],
                    dst_ref=out_ref.at[
                        pl.ds(my_pos * chunk, chunk),
                        pl.ds(h * half, half),
                    ],
                    send_sem=ag_send_sems.at[h, d - 1],
                    recv_sem=ag_recv_sems.at[h, d - 1],
                    device_id=(peer,),
                    device_id_type=pl.DeviceIdType.MESH,
                )
                rdma.start()
                ag_rdmas.append(rdma)

        for h in range(N_HALF):
            for j in range(N_DEV - 1):
                src_pos = (my_pos - j - 1) % N_DEV
                recv = pltpu.make_async_remote_copy(
                    src_ref=out_ref.at[
                        pl.ds(src_pos * chunk, chunk),
                        pl.ds(h * half, half),
                    ],
                    dst_ref=out_ref.at[
                        pl.ds(src_pos * chunk, chunk),
                        pl.ds(h * half, half),
                    ],
                    send_sem=ag_send_sems.at[h, j],
                    recv_sem=ag_recv_sems.at[h, j],
                    device_id=(my_pos,),
                    device_id_type=pl.DeviceIdType.MESH,
                )
                recv.wait_recv()

        for rdma in rs_rdmas:
            rdma.wait_send()
        for rdma in ag_rdmas:
            rdma.wait_send()


    return pl.pallas_call(
        body,
        out_shape=jax.ShapeDtypeStruct((m, n), jnp.bfloat16),
        in_specs=[
            pl.BlockSpec(memory_space=pltpu.VMEM),
            pl.BlockSpec(memory_space=pltpu.VMEM),
        ],
        out_specs=pl.BlockSpec(memory_space=pltpu.VMEM),
        scratch_shapes=[
            pltpu.VMEM((N_HALF, m, half), jnp.bfloat16),
            pltpu.VMEM((N_HALF, N_DEV - 1, chunk, half), jnp.bfloat16),
            pltpu.SemaphoreType.DMA((N_HALF, N_DEV - 1)),
            pltpu.SemaphoreType.DMA((N_HALF, N_DEV - 1)),
            pltpu.SemaphoreType.DMA((N_HALF, N_DEV - 1)),
            pltpu.SemaphoreType.DMA((N_HALF, N_DEV - 1)),
        ],
        compiler_params=pltpu.CompilerParams(collective_id=0),
    )(A, B)
